# baseline (speedup 1.0000x reference)
import functools

import jax
import jax.numpy as jnp
from jax import lax
from jax.experimental import pallas as pl
from jax.experimental.pallas import tpu as pltpu
from jax.experimental.pallas import tpu_sc as plsc

N = 10000
E = 160000
H = 128
EPS_ = 1e-8

C = 128
NSUB = 16
NCORE = 2
NCHUNK = E // C
NPAD = 10240
RPT = NPAD // NSUB
AW = 96


def _pre_body(s_ref, w1_ref, b1_ref, w2_ref, b2_ref, x_ref):
    h = jnp.dot(s_ref[...], w1_ref[...], preferred_element_type=jnp.float32)
    h = h + b1_ref[...]
    h = h * jax.nn.sigmoid(h)
    x = jnp.dot(h, w2_ref[...], preferred_element_type=jnp.float32)
    x_ref[...] = x + b2_ref[...]


def _pre_mlp(s2, W1, b1, W2, b2):
    B = 2000
    grid = (N // B,)
    return pl.pallas_call(
        _pre_body,
        grid=grid,
        in_specs=[
            pl.BlockSpec((B, H), lambda i: (i, 0)),
            pl.BlockSpec((H, H), lambda i: (0, 0)),
            pl.BlockSpec((1, H), lambda i: (0, 0)),
            pl.BlockSpec((H, 3 * H), lambda i: (0, 0)),
            pl.BlockSpec((1, 3 * H), lambda i: (0, 0)),
        ],
        out_specs=pl.BlockSpec((B, 3 * H), lambda i: (i, 0)),
        out_shape=jax.ShapeDtypeStruct((N, 3 * H), jnp.float32),
    )(s2, W1, b1.reshape(1, H), W2, b2.reshape(1, 3 * H))


def _sc_body(snd_hbm, rcvo_hbm, dirt_hbm, wij_hbm, xds2_hbm, xdv4_hbm, v4_hbm,
             ds_out, dv_out,
             snd_v, idxo_v, xj_v, vj_v, w1_v, w2_v, wds_v, d_v, out_v,
             acc, sem1, sem2, sem3, sem4):
    cid = lax.axis_index("c")
    sid = lax.axis_index("s")
    r0 = sid * RPT

    def zero_out_v():
        @plsc.parallel_loop(0, C, unroll=4)
        def zbody(e):
            for j in range(AW // 16):
                out_v[e, pl.ds(j * 16, 16)] = jnp.zeros((16,), jnp.float32)

    def zero_acc_rows():
        for m in range(RPT // C):
            pltpu.sync_copy(out_v, acc.at[pl.ds(r0 + m * C, C)])

    def flush(dst):
        pltpu.sync_copy(acc.at[pl.ds(r0, RPT)], dst)

    for p in range(2):
        hs = (2 * cid + p).astype(jnp.int32)
        zero_out_v()
        zero_acc_rows()
        plsc.subcore_barrier()

        w1off = 128 + 32 * hs
        w2off = 256 + 32 * hs

        def dv_chunk(i, carry):
            q = sid + i * NSUB

            @pl.when(q < NCHUNK)
            def _():
                e0 = q * C
                cpi = pltpu.async_copy(
                    rcvo_hbm.at[hs, pl.ds(e0, C)], idxo_v, sem3)
                cps = pltpu.async_copy(
                    snd_hbm.at[pl.ds(e0, C)], snd_v, sem4)
                cp3 = pltpu.async_copy(
                    wij_hbm.at[pl.ds(e0, C), pl.ds(w1off, 32)], w1_v, sem1)
                cp4 = pltpu.async_copy(
                    wij_hbm.at[pl.ds(e0, C), pl.ds(w2off, 32)], w2_v, sem1)
                cp5 = pltpu.async_copy(
                    dirt_hbm.at[:, pl.ds(e0, C)], d_v, sem1)
                cpi.wait()
                cp1 = pltpu.async_copy(xdv4_hbm.at[idxo_v], xj_v, sem2)
                cp2 = pltpu.async_copy(v4_hbm.at[idxo_v], vj_v, sem2)
                cp3.wait()
                cp4.wait()
                cp5.wait()
                cp1.wait()
                cp2.wait()

                @plsc.parallel_loop(0, C, unroll=8)
                def ebody(e):
                    esplat = jnp.full((16,), e, jnp.int32)
                    d0 = plsc.load_gather(
                        d_v, [jnp.zeros((16,), jnp.int32), esplat])
                    d1 = plsc.load_gather(
                        d_v, [jnp.full((16,), 1, jnp.int32), esplat])
                    d2 = plsc.load_gather(
                        d_v, [jnp.full((16,), 2, jnp.int32), esplat])
                    for j in range(2):
                        sl = pl.ds(j * 16, 16)
                        dv1 = w1_v[e, sl] * xj_v[e, sl]
                        dv2 = w2_v[e, sl] * xj_v[e, pl.ds(32 + j * 16, 16)]
                        for k, dk in ((0, d0), (1, d1), (2, d2)):
                            slk = pl.ds(k * 32 + j * 16, 16)
                            out_v[e, slk] = dv1 * dk + dv2 * vj_v[e, slk]
                cps.wait()
                pltpu.sync_copy(out_v, acc.at[snd_v], add=True)
            return carry

        lax.fori_loop(0, (NCHUNK + NSUB - 1) // NSUB, dv_chunk, 0)
        plsc.subcore_barrier()
        flush(dv_out.at[hs, pl.ds(r0, RPT)])

    zero_out_v()
    zero_acc_rows()
    plsc.subcore_barrier()

    wdoff = 64 * cid

    def ds_chunk(i, carry):
        q = sid + i * NSUB

        @pl.when(q < NCHUNK)
        def _():
            e0 = q * C
            cpi = pltpu.async_copy(
                rcvo_hbm.at[cid, pl.ds(e0, C)], idxo_v, sem3)
            cps = pltpu.async_copy(snd_hbm.at[pl.ds(e0, C)], snd_v, sem4)
            cp2 = pltpu.async_copy(
                wij_hbm.at[pl.ds(e0, C), pl.ds(wdoff, 64)], wds_v, sem1)
            cpi.wait()
            cp1 = pltpu.async_copy(xds2_hbm.at[idxo_v], xj_v, sem2)
            cp2.wait()
            cp1.wait()

            @plsc.parallel_loop(0, C, unroll=4)
            def ebody(e):
                for j in range(4):
                    sl = pl.ds(j * 16, 16)
                    out_v[e, sl] = wds_v[e, sl] * xj_v[e, sl]
            cps.wait()
            pltpu.sync_copy(out_v, acc.at[snd_v], add=True)
        return carry

    lax.fori_loop(0, (NCHUNK + NSUB - 1) // NSUB, ds_chunk, 0)
    plsc.subcore_barrier()
    flush(ds_out.at[cid, pl.ds(r0, RPT)])


def _sc_edge(snd, rcvo, dirt, wij, xds2, xdv4, v4):
    mesh = plsc.VectorSubcoreMesh(core_axis_name="c", subcore_axis_name="s")
    f32 = jnp.float32
    kern = pl.kernel(
        _sc_body,
        out_type=(
            jax.ShapeDtypeStruct((NCORE, NPAD, AW), f32),
            jax.ShapeDtypeStruct((4, NPAD, AW), f32),
        ),
        mesh=mesh,
        compiler_params=pltpu.CompilerParams(
            use_tc_tiling_on_sc=False, needs_layout_passes=False),
        scratch_types=[
            pltpu.VMEM((C,), jnp.int32),
            pltpu.VMEM((C,), jnp.int32),
            pltpu.VMEM((C, 64), f32),
            pltpu.VMEM((C, AW), f32),
            pltpu.VMEM((C, 32), f32),
            pltpu.VMEM((C, 32), f32),
            pltpu.VMEM((C, 64), f32),
            pltpu.VMEM((3, C), f32),
            pltpu.VMEM((C, AW), f32),
            pltpu.VMEM_SHARED((NPAD, AW), f32),
            pltpu.SemaphoreType.DMA,
            pltpu.SemaphoreType.DMA,
            pltpu.SemaphoreType.DMA,
            pltpu.SemaphoreType.DMA,
        ],
    )
    return kern(snd, rcvo, dirt, wij, xds2, xdv4, v4)


def _post_body(s_ref, ds_ref, dv0_ref, dv1_ref, dv2_ref,
               v0_ref, v1_ref, v2_ref,
               wv_ref, wm1_ref, bm1_ref, wm2_ref, bm2_ref,
               so_ref, v0o_ref, v1o_ref, v2o_ref):
    clip = lambda t: jnp.clip(t, -100.0, 100.0)
    s1 = s_ref[...] + clip(ds_ref[...])
    va = v0_ref[...] + clip(dv0_ref[...])
    vb = v1_ref[...] + clip(dv1_ref[...])
    vc = v2_ref[...] + clip(dv2_ref[...])
    wv = wv_ref[...]
    vma = jnp.dot(va, wv, preferred_element_type=jnp.float32)
    vmb = jnp.dot(vb, wv, preferred_element_type=jnp.float32)
    vmc = jnp.dot(vc, wv, preferred_element_type=jnp.float32)
    vla, vra = vma[:, :H], vma[:, H:]
    vlb, vrb = vmb[:, :H], vmb[:, H:]
    vlc, vrc = vmc[:, :H], vmc[:, H:]
    v_norm = jnp.sqrt(vra * vra + vrb * vrb + vrc * vrc + EPS_)
    ts = jnp.concatenate([s1, v_norm], axis=-1)
    h = jnp.dot(ts, wm1_ref[...], preferred_element_type=jnp.float32)
    h = h + bm1_ref[...]
    h = h * jax.nn.sigmoid(h)
    h2 = jnp.dot(h, wm2_ref[...], preferred_element_type=jnp.float32)
    h2 = h2 + bm2_ref[...]
    ds2, dvg, dsv = h2[:, :H], h2[:, H:2 * H], h2[:, 2 * H:]
    dot_lr = vra * vla + vrb * vlb + vrc * vlc
    so_ref[...] = s1 + clip(ds2 + dsv * dot_lr)
    v0o_ref[...] = va + clip(vla * dvg)
    v1o_ref[...] = vb + clip(vlb * dvg)
    v2o_ref[...] = vc + clip(vlc * dvg)


def _post_update(s2, ds, dv0, dv1, dv2, v0, v1, v2,
                 W_vmix, W_mix1, b_mix1, W_mix2, b_mix2):
    B = 2000
    grid = (N // B,)
    node = lambda: pl.BlockSpec((B, H), lambda i: (i, 0))
    full = lambda r, c: pl.BlockSpec((r, c), lambda i: (0, 0))
    outs = [jax.ShapeDtypeStruct((N, H), jnp.float32)] * 4
    return pl.pallas_call(
        _post_body,
        grid=grid,
        in_specs=[node() for _ in range(8)] + [
            full(H, 2 * H), full(2 * H, H), full(1, H),
            full(H, 3 * H), full(1, 3 * H),
        ],
        out_specs=[node() for _ in range(4)],
        out_shape=outs,
    )(s2, ds, dv0, dv1, dv2, v0, v1, v2,
      W_vmix, W_mix1, b_mix1.reshape(1, H), W_mix2, b_mix2.reshape(1, 3 * H))


def kernel(s, v, dir_ij, Wij, senders, receivers,
           W_int1, b_int1, W_int2, b_int2,
           W_vmix, W_mix1, b_mix1, W_mix2, b_mix2):
    s2 = s[:, 0, :]
    x = _pre_mlp(s2, W_int1, b_int1, W_int2, b_int2)

    xdv4 = jnp.concatenate([
        jnp.concatenate(
            [x[:, H + 32 * hs:H + 32 * hs + 32],
             x[:, 2 * H + 32 * hs:2 * H + 32 * hs + 32]], axis=1)
        for hs in range(4)
    ], axis=0)
    v4 = jnp.concatenate([
        v[:, :, 32 * hs:32 * hs + 32].reshape(N, AW) for hs in range(4)
    ], axis=0)
    xds2 = jnp.concatenate([x[:, :64], x[:, 64:H]], axis=0)
    dirt = dir_ij.T
    wij2 = Wij.reshape(E, 3 * H)
    rcvo = receivers[None, :] + (jnp.arange(4, dtype=jnp.int32) * N)[:, None]

    ds_out, dv_out = _sc_edge(senders, rcvo, dirt, wij2, xds2, xdv4, v4)

    ds = jnp.concatenate([ds_out[0, :N, :64], ds_out[1, :N, :64]], axis=1)
    dvs = [
        jnp.concatenate([dv_out[hs, :N, k * 32:(k + 1) * 32]
                         for hs in range(4)], axis=1)
        for k in range(3)
    ]

    so, v0o, v1o, v2o = _post_update(
        s2, ds, dvs[0], dvs[1], dvs[2],
        v[:, 0, :], v[:, 1, :], v[:, 2, :],
        W_vmix, W_mix1, b_mix1, W_mix2, b_mix2)

    return so[:, None, :], jnp.stack([v0o, v1o, v2o], axis=1)


# speedup vs baseline: 1.3033x; 1.3033x over previous
import functools

import jax
import jax.numpy as jnp
from jax import lax
from jax.experimental import pallas as pl
from jax.experimental.pallas import tpu as pltpu
from jax.experimental.pallas import tpu_sc as plsc

N = 10000
E = 160000
H = 128
EPS_ = 1e-8

C = 80
NSUB = 16
NCORE = 2
NCHUNK = E // C
ITERS = NCHUNK // NSUB
NPAD = 10240
RPT = NPAD // NSUB
AW = 96


def _pre_body(s_ref, w1_ref, b1_ref, w2_ref, b2_ref, x_ref):
    h = jnp.dot(s_ref[...], w1_ref[...], preferred_element_type=jnp.float32)
    h = h + b1_ref[...]
    h = h * jax.nn.sigmoid(h)
    x = jnp.dot(h, w2_ref[...], preferred_element_type=jnp.float32)
    x_ref[...] = x + b2_ref[...]


def _pre_mlp(s2, W1, b1, W2, b2):
    B = 2000
    grid = (N // B,)
    return pl.pallas_call(
        _pre_body,
        grid=grid,
        in_specs=[
            pl.BlockSpec((B, H), lambda i: (i, 0)),
            pl.BlockSpec((H, H), lambda i: (0, 0)),
            pl.BlockSpec((1, H), lambda i: (0, 0)),
            pl.BlockSpec((H, 3 * H), lambda i: (0, 0)),
            pl.BlockSpec((1, 3 * H), lambda i: (0, 0)),
        ],
        out_specs=pl.BlockSpec((B, 3 * H), lambda i: (i, 0)),
        out_shape=jax.ShapeDtypeStruct((N, 3 * H), jnp.float32),
    )(s2, W1, b1.reshape(1, H), W2, b2.reshape(1, 3 * H))


def _sc_body(snd_hbm, rcvo_hbm, dirt_hbm, wij_hbm, xds2_hbm, xdv4_hbm, v4_hbm,
             ds_out, dv_out,
             snd_v, idxo_v, xj_v, vj_v, w1_v, w2_v, d_v, out_v,
             acc, sem_i, sem_s, sem_g, sem_w, sem_sc):
    cid = lax.axis_index("c")
    sid = lax.axis_index("s")
    r0 = sid * RPT

    def zero_out_v(b):
        @plsc.parallel_loop(0, C, unroll=4)
        def zbody(e):
            for j in range(AW // 16):
                out_v[b][e, pl.ds(j * 16, 16)] = jnp.zeros((16,), jnp.float32)

    def zero_acc_rows():
        for m in range(RPT // C):
            pltpu.sync_copy(out_v[0], acc.at[pl.ds(r0 + m * C, C)])

    def flush(dst):
        pltpu.sync_copy(acc.at[pl.ds(r0, RPT)], dst)

    def e0_of(i):
        return (sid + i * NSUB) * C

    def run_pass(idx_row, issue_rest, wait_rest, compute):
        def issue_idx(b, i):
            pltpu.async_copy(
                rcvo_hbm.at[idx_row, pl.ds(e0_of(i), C)], idxo_v[b], sem_i[b])

        def wait_idx(b):
            pltpu.make_async_copy(
                rcvo_hbm.at[0, pl.ds(0, C)], idxo_v[b], sem_i[b]).wait()

        def issue_snd(b, i):
            pltpu.async_copy(snd_hbm.at[pl.ds(e0_of(i), C)], snd_v[b], sem_s[b])

        def wait_snd(b):
            pltpu.make_async_copy(
                snd_hbm.at[pl.ds(0, C)], snd_v[b], sem_s[b]).wait()

        def issue_scatter(b):
            pltpu.async_copy(out_v[b], acc.at[snd_v[b]], sem_sc[b], add=True)

        def wait_scatter(b):
            pltpu.make_async_copy(out_v[b], acc.at[snd_v[b]], sem_sc[b]).wait()

        def stage(b, i):
            wait_idx(b)
            issue_rest(b, i)
            issue_snd(b, i)

        def body(i, b, first, do_stage, do_idx2):
            nb = 1 - b
            wait_rest(b)
            if do_idx2:
                issue_idx(b, i + 2)
            if not first:
                wait_scatter(nb)
            if do_stage:
                stage(nb, i + 1)
            compute(b)
            wait_snd(b)
            issue_scatter(b)

        issue_idx(0, 0)
        issue_idx(1, 1)
        stage(0, 0)
        body(0, 0, True, True, True)
        body(1, 1, False, True, True)

        def mbody(i2, carry):
            i = i2 * 2
            body(i, 0, False, True, True)
            body(i + 1, 1, False, True, True)
            return carry
        lax.fori_loop(1, ITERS // 2 - 1, mbody, 0)

        body(ITERS - 3, 0, False, True, True)
        body(ITERS - 2, 1, False, True, False)
        body(ITERS - 1, 0, False, False, False)
        wait_scatter(0)

    for p in range(2):
        hs = (2 * cid + p).astype(jnp.int32)
        zero_out_v(0)
        zero_acc_rows()
        plsc.subcore_barrier()

        w1off = 128 + 32 * hs
        w2off = 256 + 32 * hs

        def dv_issue_rest(b, i):
            e0 = e0_of(i)
            pltpu.async_copy(xdv4_hbm.at[idxo_v[b]], xj_v[b], sem_g[b])
            pltpu.async_copy(v4_hbm.at[idxo_v[b]], vj_v[b], sem_g[b])
            pltpu.async_copy(
                wij_hbm.at[pl.ds(e0, C), pl.ds(w1off, 32)], w1_v[b], sem_w[b])
            pltpu.async_copy(
                wij_hbm.at[pl.ds(e0, C), pl.ds(w2off, 32)], w2_v[b], sem_w[b])
            pltpu.async_copy(dirt_hbm.at[:, pl.ds(e0, C)], d_v[b], sem_w[b])

        def dv_wait_rest(b):
            pltpu.make_async_copy(xdv4_hbm.at[idxo_v[b]], xj_v[b], sem_g[b]).wait()
            pltpu.make_async_copy(v4_hbm.at[idxo_v[b]], vj_v[b], sem_g[b]).wait()
            pltpu.make_async_copy(
                wij_hbm.at[pl.ds(0, C), pl.ds(0, 32)], w1_v[b], sem_w[b]).wait()
            pltpu.make_async_copy(
                wij_hbm.at[pl.ds(0, C), pl.ds(0, 32)], w2_v[b], sem_w[b]).wait()
            pltpu.make_async_copy(
                dirt_hbm.at[:, pl.ds(0, C)], d_v[b], sem_w[b]).wait()

        def dv_compute(b):
            @plsc.parallel_loop(0, C, unroll=4)
            def ebody(e):
                esplat = jnp.full((16,), e, jnp.int32)
                d0 = plsc.load_gather(
                    d_v[b], [jnp.zeros((16,), jnp.int32), esplat])
                d1 = plsc.load_gather(
                    d_v[b], [jnp.full((16,), 1, jnp.int32), esplat])
                d2 = plsc.load_gather(
                    d_v[b], [jnp.full((16,), 2, jnp.int32), esplat])
                for j in range(2):
                    sl = pl.ds(j * 16, 16)
                    dv1 = w1_v[b][e, sl] * xj_v[b][e, sl]
                    dv2 = w2_v[b][e, sl] * xj_v[b][e, pl.ds(32 + j * 16, 16)]
                    for k, dk in ((0, d0), (1, d1), (2, d2)):
                        slk = pl.ds(k * 32 + j * 16, 16)
                        out_v[b][e, slk] = dv1 * dk + dv2 * vj_v[b][e, slk]

        run_pass(hs, dv_issue_rest, dv_wait_rest, dv_compute)
        plsc.subcore_barrier()
        flush(dv_out.at[hs, pl.ds(r0, RPT)])

    zero_out_v(0)
    zero_out_v(1)
    zero_acc_rows()
    plsc.subcore_barrier()

    wdoff = 64 * cid

    def ds_issue_rest(b, i):
        e0 = e0_of(i)
        pltpu.async_copy(xds2_hbm.at[idxo_v[b]], xj_v[b], sem_g[b])
        pltpu.async_copy(
            wij_hbm.at[pl.ds(e0, C), pl.ds(wdoff, 32)], w1_v[b], sem_w[b])
        pltpu.async_copy(
            wij_hbm.at[pl.ds(e0, C), pl.ds(wdoff + 32, 32)], w2_v[b], sem_w[b])

    def ds_wait_rest(b):
        pltpu.make_async_copy(xds2_hbm.at[idxo_v[b]], xj_v[b], sem_g[b]).wait()
        pltpu.make_async_copy(
            wij_hbm.at[pl.ds(0, C), pl.ds(0, 32)], w1_v[b], sem_w[b]).wait()
        pltpu.make_async_copy(
            wij_hbm.at[pl.ds(0, C), pl.ds(0, 32)], w2_v[b], sem_w[b]).wait()

    def ds_compute(b):
        @plsc.parallel_loop(0, C, unroll=4)
        def ebody(e):
            for j in range(2):
                sl = pl.ds(j * 16, 16)
                out_v[b][e, sl] = w1_v[b][e, sl] * xj_v[b][e, sl]
                sl2 = pl.ds(32 + j * 16, 16)
                out_v[b][e, sl2] = w2_v[b][e, sl] * xj_v[b][e, sl2]

    run_pass(cid, ds_issue_rest, ds_wait_rest, ds_compute)
    plsc.subcore_barrier()
    flush(ds_out.at[cid, pl.ds(r0, RPT)])


def _sc_edge(snd, rcvo, dirt, wij, xds2, xdv4, v4):
    mesh = plsc.VectorSubcoreMesh(core_axis_name="c", subcore_axis_name="s")
    f32 = jnp.float32

    def db(t):
        return (t, t)

    kern = pl.kernel(
        _sc_body,
        out_type=(
            jax.ShapeDtypeStruct((NCORE, NPAD, AW), f32),
            jax.ShapeDtypeStruct((4, NPAD, AW), f32),
        ),
        mesh=mesh,
        compiler_params=pltpu.CompilerParams(
            use_tc_tiling_on_sc=False, needs_layout_passes=False),
        scratch_types=[
            db(pltpu.VMEM((C,), jnp.int32)),
            db(pltpu.VMEM((C,), jnp.int32)),
            db(pltpu.VMEM((C, 64), f32)),
            db(pltpu.VMEM((C, AW), f32)),
            db(pltpu.VMEM((C, 32), f32)),
            db(pltpu.VMEM((C, 32), f32)),
            db(pltpu.VMEM((3, C), f32)),
            db(pltpu.VMEM((C, AW), f32)),
            pltpu.VMEM_SHARED((NPAD, AW), f32),
            db(pltpu.SemaphoreType.DMA),
            db(pltpu.SemaphoreType.DMA),
            db(pltpu.SemaphoreType.DMA),
            db(pltpu.SemaphoreType.DMA),
            db(pltpu.SemaphoreType.DMA),
        ],
    )
    return kern(snd, rcvo, dirt, wij, xds2, xdv4, v4)


def _post_body(s_ref, ds_ref, dv0_ref, dv1_ref, dv2_ref,
               v0_ref, v1_ref, v2_ref,
               wv_ref, wm1_ref, bm1_ref, wm2_ref, bm2_ref,
               so_ref, v0o_ref, v1o_ref, v2o_ref):
    clip = lambda t: jnp.clip(t, -100.0, 100.0)
    s1 = s_ref[...] + clip(ds_ref[...])
    va = v0_ref[...] + clip(dv0_ref[...])
    vb = v1_ref[...] + clip(dv1_ref[...])
    vc = v2_ref[...] + clip(dv2_ref[...])
    wv = wv_ref[...]
    vma = jnp.dot(va, wv, preferred_element_type=jnp.float32)
    vmb = jnp.dot(vb, wv, preferred_element_type=jnp.float32)
    vmc = jnp.dot(vc, wv, preferred_element_type=jnp.float32)
    vla, vra = vma[:, :H], vma[:, H:]
    vlb, vrb = vmb[:, :H], vmb[:, H:]
    vlc, vrc = vmc[:, :H], vmc[:, H:]
    v_norm = jnp.sqrt(vra * vra + vrb * vrb + vrc * vrc + EPS_)
    ts = jnp.concatenate([s1, v_norm], axis=-1)
    h = jnp.dot(ts, wm1_ref[...], preferred_element_type=jnp.float32)
    h = h + bm1_ref[...]
    h = h * jax.nn.sigmoid(h)
    h2 = jnp.dot(h, wm2_ref[...], preferred_element_type=jnp.float32)
    h2 = h2 + bm2_ref[...]
    ds2, dvg, dsv = h2[:, :H], h2[:, H:2 * H], h2[:, 2 * H:]
    dot_lr = vra * vla + vrb * vlb + vrc * vlc
    so_ref[...] = s1 + clip(ds2 + dsv * dot_lr)
    v0o_ref[...] = va + clip(vla * dvg)
    v1o_ref[...] = vb + clip(vlb * dvg)
    v2o_ref[...] = vc + clip(vlc * dvg)


def _post_update(s2, ds, dv0, dv1, dv2, v0, v1, v2,
                 W_vmix, W_mix1, b_mix1, W_mix2, b_mix2):
    B = 2000
    grid = (N // B,)
    node = lambda: pl.BlockSpec((B, H), lambda i: (i, 0))
    full = lambda r, c: pl.BlockSpec((r, c), lambda i: (0, 0))
    outs = [jax.ShapeDtypeStruct((N, H), jnp.float32)] * 4
    return pl.pallas_call(
        _post_body,
        grid=grid,
        in_specs=[node() for _ in range(8)] + [
            full(H, 2 * H), full(2 * H, H), full(1, H),
            full(H, 3 * H), full(1, 3 * H),
        ],
        out_specs=[node() for _ in range(4)],
        out_shape=outs,
    )(s2, ds, dv0, dv1, dv2, v0, v1, v2,
      W_vmix, W_mix1, b_mix1.reshape(1, H), W_mix2, b_mix2.reshape(1, 3 * H))


def kernel(s, v, dir_ij, Wij, senders, receivers,
           W_int1, b_int1, W_int2, b_int2,
           W_vmix, W_mix1, b_mix1, W_mix2, b_mix2):
    s2 = s[:, 0, :]
    x = _pre_mlp(s2, W_int1, b_int1, W_int2, b_int2)

    xdv4 = jnp.concatenate([
        jnp.concatenate(
            [x[:, H + 32 * hs:H + 32 * hs + 32],
             x[:, 2 * H + 32 * hs:2 * H + 32 * hs + 32]], axis=1)
        for hs in range(4)
    ], axis=0)
    v4 = jnp.concatenate([
        v[:, :, 32 * hs:32 * hs + 32].reshape(N, AW) for hs in range(4)
    ], axis=0)
    xds2 = jnp.concatenate([x[:, :64], x[:, 64:H]], axis=0)
    dirt = dir_ij.T
    wij2 = Wij.reshape(E, 3 * H)
    rcvo = receivers[None, :] + (jnp.arange(4, dtype=jnp.int32) * N)[:, None]

    ds_out, dv_out = _sc_edge(senders, rcvo, dirt, wij2, xds2, xdv4, v4)

    ds = jnp.concatenate([ds_out[0, :N, :64], ds_out[1, :N, :64]], axis=1)
    dvs = [
        jnp.concatenate([dv_out[hs, :N, k * 32:(k + 1) * 32]
                         for hs in range(4)], axis=1)
        for k in range(3)
    ]

    so, v0o, v1o, v2o = _post_update(
        s2, ds, dvs[0], dvs[1], dvs[2],
        v[:, 0, :], v[:, 1, :], v[:, 2, :],
        W_vmix, W_mix1, b_mix1, W_mix2, b_mix2)

    return so[:, None, :], jnp.stack([v0o, v1o, v2o], axis=1)


# speedup vs baseline: 1.7059x; 1.3089x over previous
import functools

import jax
import jax.numpy as jnp
from jax import lax
from jax.experimental import pallas as pl
from jax.experimental.pallas import tpu as pltpu
from jax.experimental.pallas import tpu_sc as plsc

N = 10000
E = 160000
H = 128
EPS_ = 1e-8

C = 80
NSUB = 16
NCORE = 2
NCHUNK = E // C
ITERS = NCHUNK // NSUB
NPAD = 10240
RPT = NPAD // NSUB
AW = 96


def _pre_body(s_ref, vf_ref, w1_ref, b1_ref, w2_ref, b2_ref,
              xds_ref, xdv_ref, v4_ref):
    h = jnp.dot(s_ref[...], w1_ref[...], preferred_element_type=jnp.float32)
    h = h + b1_ref[...]
    h = h * jax.nn.sigmoid(h)
    x = jnp.dot(h, w2_ref[...], preferred_element_type=jnp.float32)
    x = x + b2_ref[...]
    xds_ref[0] = x[:, 0:64]
    xds_ref[1] = x[:, 64:128]
    vf = vf_ref[...]
    for hs in range(4):
        c1, c2 = 128 + 32 * hs, 256 + 32 * hs
        xdv_ref[hs] = jnp.concatenate([x[:, c1:c1 + 32], x[:, c2:c2 + 32]],
                                      axis=1)
        v4_ref[hs] = jnp.concatenate(
            [vf[:, k * 128 + 32 * hs:k * 128 + 32 * hs + 32]
             for k in range(3)], axis=1)


def _pre_mlp(s2, vf, W1, b1, W2, b2):
    B = 2000
    grid = (N // B,)
    f32 = jnp.float32
    return pl.pallas_call(
        _pre_body,
        grid=grid,
        in_specs=[
            pl.BlockSpec((B, H), lambda i: (i, 0)),
            pl.BlockSpec((B, 3 * H), lambda i: (i, 0)),
            pl.BlockSpec((H, H), lambda i: (0, 0)),
            pl.BlockSpec((1, H), lambda i: (0, 0)),
            pl.BlockSpec((H, 3 * H), lambda i: (0, 0)),
            pl.BlockSpec((1, 3 * H), lambda i: (0, 0)),
        ],
        out_specs=[
            pl.BlockSpec((2, B, 64), lambda i: (0, i, 0)),
            pl.BlockSpec((4, B, 64), lambda i: (0, i, 0)),
            pl.BlockSpec((4, B, AW), lambda i: (0, i, 0)),
        ],
        out_shape=[
            jax.ShapeDtypeStruct((2, N, 64), f32),
            jax.ShapeDtypeStruct((4, N, 64), f32),
            jax.ShapeDtypeStruct((4, N, AW), f32),
        ],
    )(s2, vf, W1, b1.reshape(1, H), W2, b2.reshape(1, 3 * H))


def _sc_body(snd_hbm, rcvo_hbm, dirt_hbm, wij_hbm, xds2_hbm, xdv4_hbm, v4_hbm,
             ds_out, dv_out,
             snd_v, idxo_v, xj_v, vj_v, w1_v, w2_v, d_v, out_v,
             acc, sem_i, sem_s, sem_g, sem_w, sem_sc):
    cid = lax.axis_index("c")
    sid = lax.axis_index("s")
    r0 = sid * RPT

    def zero_out_v(b):
        @plsc.parallel_loop(0, C, unroll=4)
        def zbody(e):
            for j in range(AW // 16):
                out_v[b][e, pl.ds(j * 16, 16)] = jnp.zeros((16,), jnp.float32)

    def zero_acc_rows():
        for m in range(RPT // C):
            pltpu.sync_copy(out_v[0], acc.at[pl.ds(r0 + m * C, C)])

    def flush(dst):
        pltpu.sync_copy(acc.at[pl.ds(r0, RPT)], dst)

    def e0_of(i):
        return (sid + i * NSUB) * C

    def run_pass(idx_row, issue_rest, wait_rest, compute):
        def issue_idx(b, i):
            pltpu.async_copy(
                rcvo_hbm.at[idx_row, pl.ds(e0_of(i), C)], idxo_v[b], sem_i[b])

        def wait_idx(b):
            pltpu.make_async_copy(
                rcvo_hbm.at[0, pl.ds(0, C)], idxo_v[b], sem_i[b]).wait()

        def issue_snd(b, i):
            pltpu.async_copy(snd_hbm.at[pl.ds(e0_of(i), C)], snd_v[b], sem_s[b])

        def wait_snd(b):
            pltpu.make_async_copy(
                snd_hbm.at[pl.ds(0, C)], snd_v[b], sem_s[b]).wait()

        def issue_scatter(b):
            pltpu.async_copy(out_v[b], acc.at[snd_v[b]], sem_sc[b], add=True)

        def wait_scatter(b):
            pltpu.make_async_copy(out_v[b], acc.at[snd_v[b]], sem_sc[b]).wait()

        def stage(b, i):
            wait_idx(b)
            issue_rest(b, i)
            issue_snd(b, i)

        def body(i, b, first, do_stage, do_idx2):
            nb = 1 - b
            wait_rest(b)
            if do_idx2:
                issue_idx(b, i + 2)
            if not first:
                wait_scatter(nb)
            if do_stage:
                stage(nb, i + 1)
            compute(b)
            wait_snd(b)
            issue_scatter(b)

        issue_idx(0, 0)
        issue_idx(1, 1)
        stage(0, 0)
        body(0, 0, True, True, True)
        body(1, 1, False, True, True)

        def mbody(i2, carry):
            i = i2 * 2
            body(i, 0, False, True, True)
            body(i + 1, 1, False, True, True)
            return carry
        lax.fori_loop(1, ITERS // 2 - 1, mbody, 0)

        body(ITERS - 3, 0, False, True, True)
        body(ITERS - 2, 1, False, True, False)
        body(ITERS - 1, 0, False, False, False)
        wait_scatter(0)

    for p in range(2):
        hs = (2 * cid + p).astype(jnp.int32)
        zero_out_v(0)
        zero_acc_rows()
        plsc.subcore_barrier()

        w1off = 128 + 32 * hs
        w2off = 256 + 32 * hs

        def dv_issue_rest(b, i):
            e0 = e0_of(i)
            pltpu.async_copy(xdv4_hbm.at[idxo_v[b]], xj_v[b], sem_g[b])
            pltpu.async_copy(v4_hbm.at[idxo_v[b]], vj_v[b], sem_g[b])
            pltpu.async_copy(
                wij_hbm.at[pl.ds(e0, C), pl.ds(w1off, 32)], w1_v[b], sem_w[b])
            pltpu.async_copy(
                wij_hbm.at[pl.ds(e0, C), pl.ds(w2off, 32)], w2_v[b], sem_w[b])
            pltpu.async_copy(dirt_hbm.at[:, pl.ds(e0, C)], d_v[b], sem_w[b])

        def dv_wait_rest(b):
            pltpu.make_async_copy(xdv4_hbm.at[idxo_v[b]], xj_v[b], sem_g[b]).wait()
            pltpu.make_async_copy(v4_hbm.at[idxo_v[b]], vj_v[b], sem_g[b]).wait()
            pltpu.make_async_copy(
                wij_hbm.at[pl.ds(0, C), pl.ds(0, 32)], w1_v[b], sem_w[b]).wait()
            pltpu.make_async_copy(
                wij_hbm.at[pl.ds(0, C), pl.ds(0, 32)], w2_v[b], sem_w[b]).wait()
            pltpu.make_async_copy(
                dirt_hbm.at[:, pl.ds(0, C)], d_v[b], sem_w[b]).wait()

        def dv_compute(b):
            @plsc.parallel_loop(0, C, unroll=4)
            def ebody(e):
                esplat = jnp.full((16,), e, jnp.int32)
                d0 = plsc.load_gather(
                    d_v[b], [jnp.zeros((16,), jnp.int32), esplat])
                d1 = plsc.load_gather(
                    d_v[b], [jnp.full((16,), 1, jnp.int32), esplat])
                d2 = plsc.load_gather(
                    d_v[b], [jnp.full((16,), 2, jnp.int32), esplat])
                for j in range(2):
                    sl = pl.ds(j * 16, 16)
                    dv1 = w1_v[b][e, sl] * xj_v[b][e, sl]
                    dv2 = w2_v[b][e, sl] * xj_v[b][e, pl.ds(32 + j * 16, 16)]
                    for k, dk in ((0, d0), (1, d1), (2, d2)):
                        slk = pl.ds(k * 32 + j * 16, 16)
                        out_v[b][e, slk] = dv1 * dk + dv2 * vj_v[b][e, slk]

        run_pass(hs, dv_issue_rest, dv_wait_rest, dv_compute)
        plsc.subcore_barrier()
        flush(dv_out.at[hs, pl.ds(r0, RPT)])

    zero_out_v(0)
    zero_out_v(1)
    zero_acc_rows()
    plsc.subcore_barrier()

    wdoff = 64 * cid

    def ds_issue_rest(b, i):
        e0 = e0_of(i)
        pltpu.async_copy(xds2_hbm.at[idxo_v[b]], xj_v[b], sem_g[b])
        pltpu.async_copy(
            wij_hbm.at[pl.ds(e0, C), pl.ds(wdoff, 32)], w1_v[b], sem_w[b])
        pltpu.async_copy(
            wij_hbm.at[pl.ds(e0, C), pl.ds(wdoff + 32, 32)], w2_v[b], sem_w[b])

    def ds_wait_rest(b):
        pltpu.make_async_copy(xds2_hbm.at[idxo_v[b]], xj_v[b], sem_g[b]).wait()
        pltpu.make_async_copy(
            wij_hbm.at[pl.ds(0, C), pl.ds(0, 32)], w1_v[b], sem_w[b]).wait()
        pltpu.make_async_copy(
            wij_hbm.at[pl.ds(0, C), pl.ds(0, 32)], w2_v[b], sem_w[b]).wait()

    def ds_compute(b):
        @plsc.parallel_loop(0, C, unroll=4)
        def ebody(e):
            for j in range(2):
                sl = pl.ds(j * 16, 16)
                out_v[b][e, sl] = w1_v[b][e, sl] * xj_v[b][e, sl]
                sl2 = pl.ds(32 + j * 16, 16)
                out_v[b][e, sl2] = w2_v[b][e, sl] * xj_v[b][e, sl2]

    run_pass(cid, ds_issue_rest, ds_wait_rest, ds_compute)
    plsc.subcore_barrier()
    flush(ds_out.at[cid, pl.ds(r0, RPT)])


def _sc_edge(snd, rcvo, dirt, wij, xds2, xdv4, v4):
    mesh = plsc.VectorSubcoreMesh(core_axis_name="c", subcore_axis_name="s")
    f32 = jnp.float32

    def db(t):
        return (t, t)

    kern = pl.kernel(
        _sc_body,
        out_type=(
            jax.ShapeDtypeStruct((NCORE, NPAD, AW), f32),
            jax.ShapeDtypeStruct((4, NPAD, AW), f32),
        ),
        mesh=mesh,
        compiler_params=pltpu.CompilerParams(
            use_tc_tiling_on_sc=False, needs_layout_passes=False),
        scratch_types=[
            db(pltpu.VMEM((C,), jnp.int32)),
            db(pltpu.VMEM((C,), jnp.int32)),
            db(pltpu.VMEM((C, 64), f32)),
            db(pltpu.VMEM((C, AW), f32)),
            db(pltpu.VMEM((C, 32), f32)),
            db(pltpu.VMEM((C, 32), f32)),
            db(pltpu.VMEM((3, C), f32)),
            db(pltpu.VMEM((C, AW), f32)),
            pltpu.VMEM_SHARED((NPAD, AW), f32),
            db(pltpu.SemaphoreType.DMA),
            db(pltpu.SemaphoreType.DMA),
            db(pltpu.SemaphoreType.DMA),
            db(pltpu.SemaphoreType.DMA),
            db(pltpu.SemaphoreType.DMA),
        ],
    )
    return kern(snd, rcvo, dirt, wij, xds2, xdv4, v4)


def _post_body(s_ref, dsb_ref, dvb_ref, vf_ref,
               wv_ref, wm1_ref, bm1_ref, wm2_ref, bm2_ref,
               so_ref, vo_ref):
    clip = lambda t: jnp.clip(t, -100.0, 100.0)
    ds = jnp.concatenate([dsb_ref[0, :, 0:64], dsb_ref[1, :, 0:64]], axis=1)
    dvb = dvb_ref[...]
    dv = [jnp.concatenate([dvb[hs, :, k * 32:(k + 1) * 32]
                           for hs in range(4)], axis=1) for k in range(3)]
    vf = vf_ref[...]
    s1 = s_ref[...] + clip(ds)
    va = vf[:, 0:H] + clip(dv[0])
    vb = vf[:, H:2 * H] + clip(dv[1])
    vc = vf[:, 2 * H:] + clip(dv[2])
    wv = wv_ref[...]
    vma = jnp.dot(va, wv, preferred_element_type=jnp.float32)
    vmb = jnp.dot(vb, wv, preferred_element_type=jnp.float32)
    vmc = jnp.dot(vc, wv, preferred_element_type=jnp.float32)
    vla, vra = vma[:, :H], vma[:, H:]
    vlb, vrb = vmb[:, :H], vmb[:, H:]
    vlc, vrc = vmc[:, :H], vmc[:, H:]
    v_norm = jnp.sqrt(vra * vra + vrb * vrb + vrc * vrc + EPS_)
    ts = jnp.concatenate([s1, v_norm], axis=-1)
    h = jnp.dot(ts, wm1_ref[...], preferred_element_type=jnp.float32)
    h = h + bm1_ref[...]
    h = h * jax.nn.sigmoid(h)
    h2 = jnp.dot(h, wm2_ref[...], preferred_element_type=jnp.float32)
    h2 = h2 + bm2_ref[...]
    ds2, dvg, dsv = h2[:, :H], h2[:, H:2 * H], h2[:, 2 * H:]
    dot_lr = vra * vla + vrb * vlb + vrc * vlc
    so_ref[...] = s1 + clip(ds2 + dsv * dot_lr)
    vo_ref[...] = jnp.concatenate([
        va + clip(vla * dvg),
        vb + clip(vlb * dvg),
        vc + clip(vlc * dvg),
    ], axis=1)


def _post_update(s2, ds_out, dv_out, vf,
                 W_vmix, W_mix1, b_mix1, W_mix2, b_mix2):
    B = 2000
    grid = (N // B,)
    full = lambda r, c: pl.BlockSpec((r, c), lambda i: (0, 0))
    f32 = jnp.float32
    return pl.pallas_call(
        _post_body,
        grid=grid,
        in_specs=[
            pl.BlockSpec((B, H), lambda i: (i, 0)),
            pl.BlockSpec((2, B, AW), lambda i: (0, i, 0)),
            pl.BlockSpec((4, B, AW), lambda i: (0, i, 0)),
            pl.BlockSpec((B, 3 * H), lambda i: (i, 0)),
            full(H, 2 * H), full(2 * H, H), full(1, H),
            full(H, 3 * H), full(1, 3 * H),
        ],
        out_specs=[
            pl.BlockSpec((B, H), lambda i: (i, 0)),
            pl.BlockSpec((B, 3 * H), lambda i: (i, 0)),
        ],
        out_shape=[
            jax.ShapeDtypeStruct((N, H), f32),
            jax.ShapeDtypeStruct((N, 3 * H), f32),
        ],
    )(s2, ds_out, dv_out, vf,
      W_vmix, W_mix1, b_mix1.reshape(1, H), W_mix2, b_mix2.reshape(1, 3 * H))


def kernel(s, v, dir_ij, Wij, senders, receivers,
           W_int1, b_int1, W_int2, b_int2,
           W_vmix, W_mix1, b_mix1, W_mix2, b_mix2):
    s2 = s.reshape(N, H)
    vf = v.reshape(N, 3 * H)
    xds2o, xdv4o, v4o = _pre_mlp(s2, vf, W_int1, b_int1, W_int2, b_int2)

    dirt = dir_ij.T
    wij2 = Wij.reshape(E, 3 * H)
    rcvo = receivers[None, :] + (jnp.arange(4, dtype=jnp.int32) * N)[:, None]

    ds_out, dv_out = _sc_edge(
        senders, rcvo, dirt, wij2,
        xds2o.reshape(2 * N, 64), xdv4o.reshape(4 * N, 64),
        v4o.reshape(4 * N, AW))

    so, vo = _post_update(s2, ds_out, dv_out, vf,
                          W_vmix, W_mix1, b_mix1, W_mix2, b_mix2)

    return so.reshape(N, 1, H), vo.reshape(N, 3, H)
